# baseline (speedup 1.0000x reference)
import functools

import jax
import jax.numpy as jnp
from jax import lax
from jax.experimental import pallas as pl
from jax.experimental.pallas import tpu as pltpu
from jax.experimental.pallas import tpu_sc as plsc

B = 16384
D = 64
NC = 2
NS = 16
NW = NC * NS
BPW = B // NW
CH = 128
NCH = BPW // CH
G = 16
NG = BPW // G


def _sc_body(src_hbm, trg_hbm, emb_hbm, out_hbm,
             idx_s, idx_t, rows_s, rows_t, out_v, sem):
    wid = lax.axis_index("s") * NC + lax.axis_index("c")
    base_row = wid * NCH

    pltpu.sync_copy(src_hbm.at[pl.ds(base_row, NCH)], idx_s)
    pltpu.sync_copy(trg_hbm.at[pl.ds(base_row, NCH)], idx_t)

    copies = []
    for j in range(NCH):
        copies.append(pltpu.async_copy(
            emb_hbm.at[idx_s.at[j]], rows_s.at[pl.ds(j * CH, CH)], sem))
        copies.append(pltpu.async_copy(
            emb_hbm.at[idx_t.at[j]], rows_t.at[pl.ds(j * CH, CH)], sem))
    for c in copies:
        c.wait()

    lane = lax.iota(jnp.int32, G)

    def group_body(g, carry):
        res = jnp.zeros((G,), jnp.float32)
        for j in range(G):
            p = g * G + j
            acc = jnp.zeros((16,), jnp.float32)
            for c in range(D // 16):
                s = rows_s[p, pl.ds(c * 16, 16)]
                t = rows_t[p, pl.ds(c * 16, 16)]
                acc = acc + s * t
            res = jnp.where(lane == j, jnp.sum(acc), res)
        out_v[pl.ds(g * G, G)] = res
        return carry

    lax.fori_loop(0, NG, group_body, 0)

    pltpu.sync_copy(out_v, out_hbm.at[pl.ds(wid * BPW, BPW)])


_sc_kernel = functools.partial(
    pl.kernel,
    out_type=jax.ShapeDtypeStruct((B,), jnp.float32),
    mesh=plsc.VectorSubcoreMesh(core_axis_name="c", subcore_axis_name="s"),
    compiler_params=pltpu.CompilerParams(
        needs_layout_passes=False, use_tc_tiling_on_sc=False),
    scratch_types=[
        pltpu.VMEM((NCH, CH), jnp.int32),
        pltpu.VMEM((NCH, CH), jnp.int32),
        pltpu.VMEM((BPW, D), jnp.float32),
        pltpu.VMEM((BPW, D), jnp.float32),
        pltpu.VMEM((BPW,), jnp.float32),
        pltpu.SemaphoreType.DMA,
    ],
)(_sc_body)


def kernel(network, src, trg, emb):
    src32 = src.astype(jnp.int32).reshape(NW * NCH, CH)
    trg32 = trg.astype(jnp.int32).reshape(NW * NCH, CH)
    return _sc_kernel(src32, trg32, emb)
